# baseline (speedup 1.0000x reference)
import functools

import jax
import jax.numpy as jnp
from jax import lax
from jax.experimental import pallas as pl
from jax.experimental.pallas import tpu as pltpu
from jax.experimental.pallas import tpu_sc as plsc

NUM_ROWS = 100005
EMBED = 32
BATCH = 16384
HIST = 200
NC = 2
NS = 16
NW = NC * NS
CB = 512
NBC = BATCH // CB
UNITS = HIST * NBC
UPW = UNITS // NW
BSH = NBC.bit_length() - 1

_mesh = plsc.VectorSubcoreMesh(core_axis_name="c", subcore_axis_name="s")


@functools.partial(
    pl.kernel,
    mesh=_mesh,
    compiler_params=pltpu.CompilerParams(
        use_tc_tiling_on_sc=False, needs_layout_passes=False
    ),
    out_type=jax.ShapeDtypeStruct(
        (HIST, EMBED // 8, BATCH // 128, 8, 128), jnp.float32
    ),
    scratch_types=[
        pltpu.VMEM((4, CB), jnp.int32),
        pltpu.VMEM((4, CB, EMBED), jnp.float32),
        pltpu.VMEM((2, EMBED // 8, CB // 128 + 1, 8, 129), jnp.float32),
        pltpu.SemaphoreType.DMA((4,)),
        pltpu.SemaphoreType.DMA((2,)),
    ],
)
def _emb_lookup(xt_hbm, table_hbm, out_hbm, idx_v, rows_v, trows_v, gsem, ssem):
    wid = lax.axis_index("s") * NC + lax.axis_index("c")
    u0 = wid * UPW
    lane = lax.iota(jnp.int32, 16)

    def unit_hb(u):
        return u >> BSH, (u & (NBC - 1)) * CB

    def fire(b, u):
        h, b0 = unit_hb(u)
        pltpu.sync_copy(xt_hbm.at[h, pl.ds(b0, CB)], idx_v.at[b])
        pltpu.async_copy(table_hbm.at[idx_v.at[b]], rows_v.at[b], gsem.at[b])

    def drain_gathers(b):
        pltpu.make_async_copy(
            table_hbm.at[pl.ds(0, CB)], rows_v.at[b], gsem.at[b]
        ).wait()

    eb_lo = lane >> 3
    eb_hi = eb_lo + 2
    ei = lane & 7

    def transpose(b, bt):
        def tbody(j, carry):
            jb = jnp.full((16,), j >> 2, jnp.int32)
            jc0 = jnp.full((16,), (j * 32) & 127, jnp.int32)
            for r in range(32):
                row = j * 32 + r
                jc = jc0 + r
                v0 = rows_v[b, row, pl.ds(0, 16)]
                plsc.store_scatter(trows_v.at[bt], [eb_lo, jb, ei, jc], v0)
                v1 = rows_v[b, row, pl.ds(16, 16)]
                plsc.store_scatter(trows_v.at[bt], [eb_hi, jb, ei, jc], v1)
            return carry

        lax.fori_loop(0, CB // 32, tbody, 0)

    def fire_store(b, u):
        h, b0 = unit_hb(u)
        pltpu.async_copy(
            trows_v.at[b, :, pl.ds(0, CB // 128), :, pl.ds(0, 128)],
            out_hbm.at[h, :, pl.ds(b0 // 128, CB // 128)],
            ssem.at[b],
        )

    def wait_store(b):
        pltpu.make_async_copy(
            trows_v.at[b, :, pl.ds(0, CB // 128), :, pl.ds(0, 128)],
            out_hbm.at[0, :, pl.ds(0, CB // 128)],
            ssem.at[b],
        ).wait()

    for b in range(4):
        fire(b, u0 + b)

    for b in range(4):
        u = u0 + b
        bt = b % 2
        drain_gathers(b)
        if b >= 2:
            wait_store(bt)
        transpose(b, bt)
        fire_store(bt, u)
        fire(b, u + 4)

    def body(j, carry):
        u = u0 + 4 * j
        for b in range(4):
            bt = b % 2
            drain_gathers(b)
            wait_store(bt)
            transpose(b, bt)
            fire_store(bt, u + b)
            fire(b, u + b + 4)
        return carry

    lax.fori_loop(1, UPW // 4 - 1, body, 0)

    for b in range(4):
        u = u0 + UPW - 4 + b
        bt = b % 2
        drain_gathers(b)
        wait_store(bt)
        transpose(b, bt)
        fire_store(bt, u)
    wait_store(0)
    wait_store(1)


def kernel(x, table):
    xt = jnp.transpose(x.astype(jnp.int32))
    out5 = _emb_lookup(xt, table)
    out = jnp.transpose(out5, (2, 4, 0, 1, 3))
    return out.reshape(BATCH, HIST, EMBED)


# speedup vs baseline: 1.0058x; 1.0058x over previous
import functools

import jax
import jax.numpy as jnp
from jax import lax
from jax.experimental import pallas as pl
from jax.experimental.pallas import tpu as pltpu
from jax.experimental.pallas import tpu_sc as plsc

NUM_ROWS = 100005
EMBED = 32
BATCH = 16384
HIST = 200
NC = 2
NS = 16
NW = NC * NS
CB = 512
NBC = BATCH // CB
UNITS = HIST * NBC
UPW = UNITS // NW
BSH = NBC.bit_length() - 1

_mesh = plsc.VectorSubcoreMesh(core_axis_name="c", subcore_axis_name="s")


@functools.partial(
    pl.kernel,
    mesh=_mesh,
    compiler_params=pltpu.CompilerParams(
        use_tc_tiling_on_sc=False, needs_layout_passes=False
    ),
    out_type=jax.ShapeDtypeStruct(
        (HIST, EMBED // 8, BATCH // 128, 8, 128), jnp.float32
    ),
    scratch_types=[
        pltpu.VMEM((4, CB), jnp.int32),
        pltpu.VMEM((4, CB, EMBED), jnp.float32),
        pltpu.VMEM((2, EMBED // 8, CB // 128 + 1, 8, 129), jnp.float32),
        pltpu.SemaphoreType.DMA((4,)),
        pltpu.SemaphoreType.DMA((2,)),
    ],
)
def _emb_lookup(xt_hbm, table_hbm, out_hbm, idx_v, rows_v, trows_v, gsem, ssem):
    wid = lax.axis_index("s") * NC + lax.axis_index("c")
    u0 = wid * UPW
    lane = lax.iota(jnp.int32, 16)

    def unit_hb(u):
        return u >> BSH, (u & (NBC - 1)) * CB

    def fire(b, u):
        h, b0 = unit_hb(u)
        pltpu.sync_copy(xt_hbm.at[h, pl.ds(b0, CB)], idx_v.at[b])
        pltpu.async_copy(table_hbm.at[idx_v.at[b]], rows_v.at[b], gsem.at[b])

    def drain_gathers(b):
        pltpu.make_async_copy(
            table_hbm.at[pl.ds(0, CB)], rows_v.at[b], gsem.at[b]
        ).wait()

    eb_lo = lane >> 3
    eb_hi = eb_lo + 2
    ei = lane & 7

    def transpose(b, bt):
        def tbody(j, carry):
            jb = jnp.full((16,), j >> 3, jnp.int32)
            jc0 = jnp.full((16,), (j * 16) & 127, jnp.int32)
            for r in range(16):
                row = j * 16 + r
                jc = jc0 + r
                v0 = rows_v[b, row, pl.ds(0, 16)]
                plsc.store_scatter(trows_v.at[bt], [eb_lo, jb, ei, jc], v0)
                v1 = rows_v[b, row, pl.ds(16, 16)]
                plsc.store_scatter(trows_v.at[bt], [eb_hi, jb, ei, jc], v1)
            return carry

        lax.fori_loop(0, CB // 16, tbody, 0)

    def fire_store(b, u):
        h, b0 = unit_hb(u)
        pltpu.async_copy(
            trows_v.at[b, :, pl.ds(0, CB // 128), :, pl.ds(0, 128)],
            out_hbm.at[h, :, pl.ds(b0 // 128, CB // 128)],
            ssem.at[b],
        )

    def wait_store(b):
        pltpu.make_async_copy(
            trows_v.at[b, :, pl.ds(0, CB // 128), :, pl.ds(0, 128)],
            out_hbm.at[0, :, pl.ds(0, CB // 128)],
            ssem.at[b],
        ).wait()

    for b in range(4):
        fire(b, u0 + b)

    for b in range(4):
        u = u0 + b
        bt = b % 2
        drain_gathers(b)
        if b >= 2:
            wait_store(bt)
        transpose(b, bt)
        fire_store(bt, u)
        fire(b, u + 4)

    def body(j, carry):
        u = u0 + 4 * j
        for b in range(4):
            bt = b % 2
            drain_gathers(b)
            wait_store(bt)
            transpose(b, bt)
            fire_store(bt, u + b)
            fire(b, u + b + 4)
        return carry

    lax.fori_loop(1, UPW // 4 - 1, body, 0)

    for b in range(4):
        u = u0 + UPW - 4 + b
        bt = b % 2
        drain_gathers(b)
        wait_store(bt)
        transpose(b, bt)
        fire_store(bt, u)
    wait_store(0)
    wait_store(1)


def kernel(x, table):
    xt = jnp.transpose(x.astype(jnp.int32))
    out5 = _emb_lookup(xt, table)
    out = jnp.transpose(out5, (2, 4, 0, 1, 3))
    return out.reshape(BATCH, HIST, EMBED)
